# baseline (speedup 1.0000x reference)
import functools

import jax
import jax.numpy as jnp
from jax import lax
from jax.experimental import pallas as pl
from jax.experimental.pallas import tpu as pltpu


def _gate(pooled, w1_ref, w2t_ref):
    h = lax.dot_general(
        pooled.astype(w1_ref.dtype), w1_ref[...],
        dimension_numbers=(((1,), (1,)), ((), ())),
        preferred_element_type=jnp.float32,
        precision=lax.Precision.HIGHEST)
    h = jnp.where(h >= 0, h, 0.01 * h)
    s = lax.dot_general(
        h.astype(w2t_ref.dtype), w2t_ref[...],
        dimension_numbers=(((1,), (0,)), ((), ())),
        preferred_element_type=jnp.float32,
        precision=lax.Precision.HIGHEST)
    return jax.nn.sigmoid(s)


def _se_stream(x_hbm, w1_ref, w2t_ref, o_hbm, in_buf, out_buf, in_sem,
               out_sem, *, steps, bt, nbuf, inv_hw):
    def in_copy(s, idx):
        return pltpu.make_async_copy(
            x_hbm.at[pl.ds(s * bt, bt)], in_buf.at[idx], in_sem.at[idx])

    def out_copy(s, idx):
        return pltpu.make_async_copy(
            out_buf.at[idx], o_hbm.at[pl.ds(s * bt, bt)], out_sem.at[idx])

    for s in range(min(nbuf - 1, steps)):
        in_copy(s, s % nbuf).start()

    def step(s, carry):
        idx = lax.rem(s, nbuf)
        nxt = s + nbuf - 1

        @pl.when(nxt < steps)
        def _():
            in_copy(jnp.minimum(nxt, steps - 1), lax.rem(nxt, nbuf)).start()

        in_copy(s, idx).wait()

        @pl.when(s >= nbuf)
        def _():
            out_copy(jnp.maximum(s - nbuf, 0), idx).wait()

        xv = in_buf[idx]
        col = jnp.sum(xv, axis=1, dtype=jnp.float32)
        pooled = jnp.sum(col, axis=1, dtype=jnp.float32) * inv_hw
        g = _gate(pooled, w1_ref, w2t_ref).astype(xv.dtype)
        out_buf[idx] = xv * g[:, None, None, :]
        out_copy(s, idx).start()
        return carry

    lax.fori_loop(0, steps, step, 0)

    for s in range(max(steps - nbuf, 0), steps):
        out_copy(s, s % nbuf).wait()


def kernel(x, w1, w2):
    B, C, H, W = x.shape
    Cr = w1.shape[0]
    xt = jnp.transpose(x, (0, 2, 3, 1))

    itemsize = jnp.dtype(x.dtype).itemsize
    img_bytes = H * W * C * itemsize
    nbuf = 8
    budget = 54 << 20
    bt = 1
    for cand in range(1, B + 1):
        if B % cand == 0 and 2 * nbuf * cand * img_bytes <= budget:
            bt = cand
    steps = B // bt

    body = functools.partial(
        _se_stream, steps=steps, bt=bt, nbuf=nbuf, inv_hw=1.0 / (H * W))
    out_t = pl.pallas_call(
        body,
        out_shape=jax.ShapeDtypeStruct((B, H, W, C), x.dtype),
        in_specs=[
            pl.BlockSpec(memory_space=pl.ANY),
            pl.BlockSpec((Cr, C), lambda: (0, 0)),
            pl.BlockSpec((Cr, C), lambda: (0, 0)),
        ],
        out_specs=pl.BlockSpec(memory_space=pl.ANY),
        scratch_shapes=[
            pltpu.VMEM((nbuf, bt, H, W, C), x.dtype),
            pltpu.VMEM((nbuf, bt, H, W, C), x.dtype),
            pltpu.SemaphoreType.DMA((nbuf,)),
            pltpu.SemaphoreType.DMA((nbuf,)),
        ],
        compiler_params=pltpu.CompilerParams(
            vmem_limit_bytes=(62 << 20)),
    )(xt, w1, w2.T)
    return jnp.transpose(out_t, (0, 3, 1, 2))


# speedup vs baseline: 1.0184x; 1.0184x over previous
import functools

import jax
import jax.numpy as jnp
from jax import lax
from jax.experimental import pallas as pl
from jax.experimental.pallas import tpu as pltpu


def _roundup(n, m):
    return ((n + m - 1) // m) * m


def _se_body(x_ref, w1_ref, w2t_ref, o_ref, *, inv_hw):
    xv = x_ref[...]

    col = jnp.sum(xv, axis=1, dtype=jnp.float32)
    pooled = jnp.sum(col, axis=1, dtype=jnp.float32) * inv_hw

    h = lax.dot_general(
        pooled.astype(w1_ref.dtype), w1_ref[...],
        dimension_numbers=(((1,), (1,)), ((), ())),
        preferred_element_type=jnp.float32,
        precision=lax.Precision.HIGHEST)
    h = jnp.where(h >= 0, h, 0.01 * h)
    s = lax.dot_general(
        h.astype(w2t_ref.dtype), w2t_ref[...],
        dimension_numbers=(((1,), (0,)), ((), ())),
        preferred_element_type=jnp.float32,
        precision=lax.Precision.HIGHEST)
    gate = jax.nn.sigmoid(s).astype(o_ref.dtype)

    o_ref[...] = xv * gate[:, None, None, :]


def _pick_batch_tile(B, bytes_per_image, budget_bytes):
    best = 1
    for bt in range(1, B + 1):
        if B % bt:
            continue
        steps = B // bt
        if steps % 2 and steps != 1:
            continue
        if 4 * bt * bytes_per_image > budget_bytes:
            break
        best = bt
    return best


def kernel(x, w1, w2):
    B, C, H, W = x.shape
    Cr = w1.shape[0]
    xt = jnp.transpose(x, (0, 2, 3, 1))

    itemsize = jnp.dtype(x.dtype).itemsize
    sub = 8 * max(1, 4 // itemsize)
    bytes_per_image = H * _roundup(W, sub) * _roundup(C, 128) * itemsize

    budget = 56 << 20
    Bt = _pick_batch_tile(B, bytes_per_image, budget)

    out_t = pl.pallas_call(
        functools.partial(_se_body, inv_hw=1.0 / (H * W)),
        out_shape=jax.ShapeDtypeStruct((B, H, W, C), x.dtype),
        grid=(B // Bt,),
        in_specs=[
            pl.BlockSpec((Bt, H, W, C), lambda b: (b, 0, 0, 0)),
            pl.BlockSpec((Cr, C), lambda b: (0, 0)),
            pl.BlockSpec((Cr, C), lambda b: (0, 0)),
        ],
        out_specs=pl.BlockSpec((Bt, H, W, C), lambda b: (b, 0, 0, 0)),
        compiler_params=pltpu.CompilerParams(
            dimension_semantics=("parallel",),
            vmem_limit_bytes=(62 << 20)),
    )(xt, w1, w2.T)
    return jnp.transpose(out_t, (0, 3, 1, 2))


# speedup vs baseline: 1.0196x; 1.0012x over previous
import functools

import jax
import jax.numpy as jnp
from jax import lax
from jax.experimental import pallas as pl
from jax.experimental.pallas import tpu as pltpu


def _roundup(n, m):
    return ((n + m - 1) // m) * m


def _se_body(x_ref, w1_ref, w2t_ref, o_ref, *, inv_hw):
    xv = x_ref[...]

    pooled = jnp.sum(xv, axis=1, dtype=jnp.float32) * inv_hw

    h = lax.dot_general(
        pooled.astype(w1_ref.dtype), w1_ref[...],
        dimension_numbers=(((1,), (1,)), ((), ())),
        preferred_element_type=jnp.float32,
        precision=lax.Precision.HIGHEST)
    h = jnp.where(h >= 0, h, 0.01 * h)
    s = lax.dot_general(
        h.astype(w2t_ref.dtype), w2t_ref[...],
        dimension_numbers=(((1,), (0,)), ((), ())),
        preferred_element_type=jnp.float32,
        precision=lax.Precision.HIGHEST)
    gate = jax.nn.sigmoid(s).astype(o_ref.dtype)

    o_ref[...] = xv * gate[:, None, :]


def _pick_batch_tile(B, bytes_per_image, budget_bytes):
    best = 1
    for bt in range(1, B + 1):
        if B % bt:
            continue
        steps = B // bt
        if steps % 2 and steps != 1:
            continue
        if 4 * bt * bytes_per_image > budget_bytes:
            break
        best = bt
    return best


def kernel(x, w1, w2):
    B, C, H, W = x.shape
    Cr = w1.shape[0]
    HW = H * W
    xt = jnp.transpose(x, (0, 2, 3, 1)).reshape(B, HW, C)

    itemsize = jnp.dtype(x.dtype).itemsize
    sub = 8 * max(1, 4 // itemsize)
    bytes_per_image = _roundup(HW, sub) * _roundup(C, 128) * itemsize

    budget = 56 << 20
    Bt = _pick_batch_tile(B, bytes_per_image, budget)

    out_t = pl.pallas_call(
        functools.partial(_se_body, inv_hw=1.0 / HW),
        out_shape=jax.ShapeDtypeStruct((B, HW, C), x.dtype),
        grid=(B // Bt,),
        in_specs=[
            pl.BlockSpec((Bt, HW, C), lambda b: (b, 0, 0)),
            pl.BlockSpec((Cr, C), lambda b: (0, 0)),
            pl.BlockSpec((Cr, C), lambda b: (0, 0)),
        ],
        out_specs=pl.BlockSpec((Bt, HW, C), lambda b: (b, 0, 0)),
        compiler_params=pltpu.CompilerParams(
            dimension_semantics=("parallel",),
            vmem_limit_bytes=(62 << 20)),
    )(xt, w1, w2.T)
    return jnp.transpose(out_t.reshape(B, H, W, C), (0, 3, 1, 2))
